# baseline (speedup 1.0000x reference)
import functools

import jax
import jax.numpy as jnp
from jax import lax
from jax.experimental import pallas as pl
from jax.experimental.pallas import tpu as pltpu
from jax.experimental.pallas import tpu_sc as plsc

_VOCAB = 1000
_B = 1024
_L = 50
_LP = 56
_CK = 128
_NW = 32
_BPW = _B // _NW

_mesh = plsc.VectorSubcoreMesh(core_axis_name="c", subcore_axis_name="s")


@functools.partial(
    pl.kernel,
    mesh=_mesh,
    out_type=[
        jax.ShapeDtypeStruct((_B, _L, _VOCAB), jnp.float32),
        jax.ShapeDtypeStruct((_B, _LP, 128), jnp.float32),
    ],
    scratch_types=[
        pltpu.VMEM((_BPW * _CK,), jnp.int32),
        pltpu.VMEM((2, _LP, 8, 128), jnp.float32),
        pltpu.SemaphoreType.DMA,
        pltpu.SemaphoreType.DMA,
        pltpu.SemaphoreType.DMA,
        pltpu.SemaphoreType.DMA,
    ],
)
def _embed(e_hbm, s_hbm, out_hbm, e7_hbm, idx_v, rows_v, gs0, gs1, ws0, ws1):
    cid = lax.axis_index("c")
    sid = lax.axis_index("s")
    wid = sid * 2 + cid
    b0 = wid * _BPW

    pltpu.sync_copy(e_hbm.at[pl.ds(wid * _BPW * _CK, _BPW * _CK)], idx_v)

    def gather(q, buf):
        pltpu.async_copy(
            s_hbm.at[idx_v.at[pl.ds(q * _CK, _LP)]],
            rows_v.at[buf],
            gs0 if buf == 0 else gs1,
        )

    def gwait(buf):
        pltpu.make_async_copy(
            s_hbm.at[idx_v.at[pl.ds(0, _LP)]],
            rows_v.at[buf],
            gs0 if buf == 0 else gs1,
        ).wait()

    def _write_list(q, buf, drain):
        b = b0 + q
        ws = ws0 if buf == 0 else ws1
        for j in range(7):
            pairs = [
                (
                    rows_v.at[buf, pl.ds(0, 48), j],
                    out_hbm.at[b, pl.ds(0, 48), pl.ds(j * 128, 128)],
                ),
                (
                    rows_v.at[buf, pl.ds(48, 2), j],
                    out_hbm.at[b, pl.ds(48, 2), pl.ds(j * 128, 128)],
                ),
            ]
            for src, dst in pairs:
                if drain:
                    pltpu.make_async_copy(src, dst, ws).wait()
                else:
                    pltpu.async_copy(src, dst, ws)
        src = rows_v.at[buf, pl.ds(0, _LP), 7]
        dst = e7_hbm.at[b]
        if drain:
            pltpu.make_async_copy(src, dst, ws).wait()
        else:
            pltpu.async_copy(src, dst, ws)

    gather(0, 0)

    def body(p, carry):
        q0 = p * 2
        gwait(0)

        @pl.when(p > 0)
        def _():
            _write_list(q0 - 1, 1, drain=True)

        gather(q0 + 1, 1)
        _write_list(q0, 0, drain=False)

        gwait(1)
        _write_list(q0, 0, drain=True)

        @pl.when(q0 + 2 < _BPW)
        def _():
            gather(q0 + 2, 0)

        _write_list(q0 + 1, 1, drain=False)
        return carry

    lax.fori_loop(0, _BPW // 2, body, 0)
    _write_list(_BPW - 1, 1, drain=True)


def kernel(idx, targets, token_embedding_table):
    del targets
    s = jnp.pad(token_embedding_table, ((0, 0), (0, 24))).reshape(_VOCAB, 8, 128)
    idxp = jnp.pad(idx.astype(jnp.int32), ((0, 0), (0, _LP - _L)))
    e = jnp.pad(idxp, ((0, 0), (0, _CK - _LP))).reshape(-1)
    main, e7 = _embed(e, s)
    return jax.lax.dynamic_update_slice(main, e7[:, :_L, :104], (0, 0, 896))


# speedup vs baseline: 1.2321x; 1.2321x over previous
import functools

import jax
import jax.numpy as jnp
from jax import lax
from jax.experimental import pallas as pl
from jax.experimental.pallas import tpu as pltpu
from jax.experimental.pallas import tpu_sc as plsc

_VOCAB = 1000
_B = 1024
_L = 50
_NW = 32
_BPW = _B // _NW
_HALF = _L // 2
_NCHUNK = _BPW * 2
_SLAB = 64

_mesh = plsc.VectorSubcoreMesh(core_axis_name="c", subcore_axis_name="s")


@functools.partial(
    pl.kernel,
    mesh=_mesh,
    out_type=jax.ShapeDtypeStruct((_B, _L, _VOCAB), jnp.float32),
    scratch_types=[
        pltpu.VMEM((_NCHUNK, _HALF), jnp.int32),
        pltpu.VMEM((2, _HALF, _VOCAB), jnp.float32),
        pltpu.VMEM_SHARED((_VOCAB, _VOCAB), jnp.float32),
        pltpu.SemaphoreType.DMA,
        pltpu.SemaphoreType.DMA,
    ],
    compiler_params=pltpu.CompilerParams(use_tc_tiling_on_sc=False),
)
def _embed(idx_hbm, table_hbm, out_hbm, idx_v, rows_v, table_sh, sem0, sem1):
    cid = lax.axis_index("c")
    sid = lax.axis_index("s")
    wid = sid * 2 + cid
    b0 = wid * _BPW

    @pl.when(sid < 15)
    def _():
        pltpu.sync_copy(
            table_hbm.at[pl.ds(sid * _SLAB, _SLAB)],
            table_sh.at[pl.ds(sid * _SLAB, _SLAB)],
        )

    @pl.when(sid == 15)
    def _():
        pltpu.sync_copy(
            table_hbm.at[pl.ds(15 * _SLAB, _VOCAB - 15 * _SLAB)],
            table_sh.at[pl.ds(15 * _SLAB, _VOCAB - 15 * _SLAB)],
        )

    pltpu.sync_copy(idx_hbm.at[wid], idx_v)
    plsc.subcore_barrier()

    pltpu.async_copy(table_sh.at[idx_v.at[0]], rows_v.at[0], sem0)

    def body(p, carry):
        b = b0 + p
        g0 = p * 2
        pltpu.make_async_copy(table_sh.at[idx_v.at[g0]], rows_v.at[0], sem0).wait()
        pltpu.async_copy(table_sh.at[idx_v.at[g0 + 1]], rows_v.at[1], sem1)
        pltpu.sync_copy(rows_v.at[0], out_hbm.at[b, pl.ds(0, _HALF)])

        pltpu.make_async_copy(table_sh.at[idx_v.at[g0 + 1]], rows_v.at[1], sem1).wait()

        @pl.when(g0 + 2 < _NCHUNK)
        def _():
            pltpu.async_copy(table_sh.at[idx_v.at[g0 + 2]], rows_v.at[0], sem0)

        pltpu.sync_copy(rows_v.at[1], out_hbm.at[b, pl.ds(_HALF, _HALF)])
        return carry

    lax.fori_loop(0, _BPW, body, 0)


def kernel(idx, targets, token_embedding_table):
    del targets
    idx3 = idx.reshape(_NW, _NCHUNK, _HALF).astype(jnp.int32)
    return _embed(idx3, token_embedding_table)
